# Initial kernel scaffold; baseline (speedup 1.0000x reference)
import jax
import jax.numpy as jnp
from jax.experimental import pallas as pl


def kernel(x, weight):
    raise NotImplementedError("write your pallas kernel here")


# speedup vs baseline: 1.5231x; 1.5231x over previous
import functools

import jax
import jax.numpy as jnp
from jax import lax
from jax.experimental import pallas as pl
from jax.experimental.pallas import tpu as pltpu
from jax.experimental.pallas import tpu_sc as plsc

D = 32
CHUNK = 128
NW = 32


@functools.lru_cache(maxsize=None)
def _make_kernel(B):
    b_per_w = B // NW
    nch = b_per_w // CHUNK
    mesh = plsc.VectorSubcoreMesh(core_axis_name="c", subcore_axis_name="s")

    @functools.partial(
        pl.kernel,
        mesh=mesh,
        compiler_params=pltpu.CompilerParams(use_tc_tiling_on_sc=False),
        out_type=jax.ShapeDtypeStruct((B, D), jnp.float32),
        scratch_types=[
            pltpu.VMEM((nch, CHUNK), jnp.int32),
            pltpu.VMEM((CHUNK, D), jnp.float32),
            pltpu.VMEM((CHUNK, D), jnp.float32),
            pltpu.SemaphoreType.DMA,
            pltpu.SemaphoreType.DMA,
        ],
    )
    def k(x_hbm, w_hbm, out_hbm, idx_v, rows0, rows1, sem0, sem1):
        c = lax.axis_index("c")
        s = lax.axis_index("s")
        wid = s * 2 + c
        base = wid * b_per_w

        pltpu.sync_copy(x_hbm.at[wid], idx_v)

        pltpu.async_copy(w_hbm.at[idx_v.at[0]], rows0, sem0)
        pltpu.async_copy(w_hbm.at[idx_v.at[1]], rows1, sem1)

        def body(i, carry):
            j = i * 2
            for b, (rows, sem) in enumerate(((rows0, sem0), (rows1, sem1))):
                jb = j + b
                pltpu.make_async_copy(w_hbm.at[idx_v.at[jb]], rows, sem).wait()
                pltpu.sync_copy(
                    rows, out_hbm.at[pl.ds(base + jb * CHUNK, CHUNK)]
                )

                @pl.when(jb + 2 < nch)
                def _():
                    pltpu.async_copy(w_hbm.at[idx_v.at[jb + 2]], rows, sem)

            return carry

        lax.fori_loop(0, nch // 2, body, 0)

    return k


def kernel(x, weight):
    BATCH, FIELDS = x.shape
    B = BATCH * FIELDS
    x_flat = x.reshape(NW, (B // NW) // CHUNK, CHUNK).astype(jnp.int32)
    out = _make_kernel(B)(x_flat, weight)
    return out.reshape(BATCH, FIELDS, D)


# speedup vs baseline: 1.5759x; 1.0346x over previous
import functools

import jax
import jax.numpy as jnp
from jax import lax
from jax.experimental import pallas as pl
from jax.experimental.pallas import tpu as pltpu
from jax.experimental.pallas import tpu_sc as plsc

D = 32
CHUNK = 512
NW = 32


@functools.lru_cache(maxsize=None)
def _make_kernel(B):
    b_per_w = B // NW
    nch = b_per_w // CHUNK
    mesh = plsc.VectorSubcoreMesh(core_axis_name="c", subcore_axis_name="s")

    @functools.partial(
        pl.kernel,
        mesh=mesh,
        compiler_params=pltpu.CompilerParams(use_tc_tiling_on_sc=False),
        out_type=jax.ShapeDtypeStruct((B, D), jnp.float32),
        scratch_types=[
            pltpu.VMEM((nch, CHUNK), jnp.int32),
            pltpu.VMEM((CHUNK, D), jnp.float32),
            pltpu.VMEM((CHUNK, D), jnp.float32),
            pltpu.SemaphoreType.DMA,
            pltpu.SemaphoreType.DMA,
        ],
    )
    def k(x_hbm, w_hbm, out_hbm, idx_v, rows0, rows1, sem0, sem1):
        c = lax.axis_index("c")
        s = lax.axis_index("s")
        wid = s * 2 + c
        base = wid * b_per_w

        pltpu.sync_copy(x_hbm.at[wid], idx_v)

        pltpu.async_copy(w_hbm.at[idx_v.at[0]], rows0, sem0)
        pltpu.async_copy(w_hbm.at[idx_v.at[1]], rows1, sem1)

        def body(i, carry):
            j = i * 2
            for b, (rows, sem) in enumerate(((rows0, sem0), (rows1, sem1))):
                jb = j + b
                pltpu.make_async_copy(w_hbm.at[idx_v.at[jb]], rows, sem).wait()
                pltpu.sync_copy(
                    rows, out_hbm.at[pl.ds(base + jb * CHUNK, CHUNK)]
                )

                @pl.when(jb + 2 < nch)
                def _():
                    pltpu.async_copy(w_hbm.at[idx_v.at[jb + 2]], rows, sem)

            return carry

        lax.fori_loop(0, nch // 2, body, 0)

    return k


def kernel(x, weight):
    BATCH, FIELDS = x.shape
    B = BATCH * FIELDS
    x_flat = x.reshape(NW, (B // NW) // CHUNK, CHUNK).astype(jnp.int32)
    out = _make_kernel(B)(x_flat, weight)
    return out.reshape(BATCH, FIELDS, D)
